# baseline (speedup 1.0000x reference)
import functools

import jax
import jax.numpy as jnp
from jax import lax
from jax.experimental import pallas as pl
from jax.experimental.pallas import tpu as pltpu
from jax.experimental.pallas import tpu_sc as plsc


def _emb_lookup(idx_grouped, weight, *, nw, t_steps, bb, d):
    mesh = plsc.VectorSubcoreMesh(core_axis_name="c", subcore_axis_name="s")
    nbuf = 4
    rbs = d // 8

    @functools.partial(
        pl.kernel,
        out_type=jax.ShapeDtypeStruct((t_steps, rbs, nw, 8, bb), jnp.float32),
        mesh=mesh,
        scratch_types=[
            pltpu.VMEM((t_steps, bb), jnp.int32),
            pltpu.VMEM((nbuf, bb, d), jnp.float32),
            pltpu.VMEM((nbuf, d, bb + 1), jnp.float32),
        ] + [pltpu.SemaphoreType.DMA] * 8,
        compiler_params=pltpu.CompilerParams(
            use_tc_tiling_on_sc=False,
            skip_device_barrier=True,
            needs_layout_passes=False,
        ),
    )
    def emb(idx_hbm, w_hbm, out_hbm, idx_v, rows_v, slab_v, *sems):
        gsems = sems[:4]
        ssems = sems[4:]
        wid = lax.axis_index("s") * mesh.num_cores + lax.axis_index("c")
        pltpu.sync_copy(idx_hbm.at[wid], idx_v)
        for b in range(nbuf):
            pltpu.async_copy(
                w_hbm.at[idx_v.at[b]], rows_v.at[b], gsems[b]
            )

        lane = lax.iota(jnp.int32, 16)
        dv = [lane + 16 * g2 for g2 in range(d // 16)]

        @pl.loop(0, t_steps // nbuf)
        def _(g):
            t0 = g * nbuf
            for b in range(nbuf):
                t = t0 + b
                rows_f = rows_v.at[b]
                pltpu.make_async_copy(
                    w_hbm.at[idx_v.at[b]], rows_v.at[b], gsems[b]
                ).wait()
                @pl.when(t >= nbuf)
                def _():
                    for rb in range(rbs):
                        pltpu.make_async_copy(
                            slab_v.at[b, pl.ds(rb * 8, 8), pl.ds(0, bb)],
                            out_hbm.at[0, rb, 0],
                            ssems[b],
                        ).wait()

                slab2 = slab_v.at[b]

                @pl.loop(0, bb, unroll=4)
                def _(j):
                    jspl = jnp.broadcast_to(j, (16,)).astype(jnp.int32)
                    for g2 in range(d // 16):
                        vals = rows_f[j, pl.ds(16 * g2, 16)]
                        plsc.store_scatter(slab2, [dv[g2], jspl], vals)

                for rb in range(rbs):
                    pltpu.async_copy(
                        slab_v.at[b, pl.ds(rb * 8, 8), pl.ds(0, bb)],
                        out_hbm.at[t, rb, wid],
                        ssems[b],
                    )
                nt = t + nbuf

                @pl.when(nt < t_steps)
                def _():
                    pltpu.async_copy(
                        w_hbm.at[idx_v.at[nt]], rows_v.at[b], gsems[b]
                    )

        for b in range(nbuf):
            for rb in range(rbs):
                pltpu.make_async_copy(
                    slab_v.at[b, pl.ds(rb * 8, 8), pl.ds(0, bb)],
                    out_hbm.at[0, rb, 0],
                    ssems[b],
                ).wait()

    return emb(idx_grouped, weight)


def kernel(idx, weight):
    b, t = idx.shape
    v, d = weight.shape
    nw = 32
    bb = b // nw
    assert bb * nw == b and d % 8 == 0 and bb % 16 == 0

    idx_grouped = (
        idx.astype(jnp.int32).reshape(nw, bb, t).transpose(0, 2, 1)
    )
    out5 = _emb_lookup(idx_grouped, weight, nw=nw, t_steps=t, bb=bb, d=d)
    return out5.transpose(2, 4, 0, 1, 3).reshape(b, t, d)


# speedup vs baseline: 1.0489x; 1.0489x over previous
import functools

import jax
import jax.numpy as jnp
from jax import lax
from jax.experimental import pallas as pl
from jax.experimental.pallas import tpu as pltpu
from jax.experimental.pallas import tpu_sc as plsc

_LANES = 128


def _emb_lookup(idx_grouped, weight, *, nw, n_ch, ch, d):
    n = nw * n_ch * ch
    per_w = n_ch * ch
    mesh = plsc.VectorSubcoreMesh(core_axis_name="c", subcore_axis_name="s")
    nbuf = 2

    @functools.partial(
        pl.kernel,
        out_type=jax.ShapeDtypeStruct((n, _LANES), jnp.float32),
        mesh=mesh,
        scratch_types=[
            pltpu.VMEM((n_ch, ch), jnp.int32),
            pltpu.VMEM((nbuf, ch, d), jnp.float32),
            pltpu.SemaphoreType.DMA,
            pltpu.SemaphoreType.DMA,
        ],
        compiler_params=pltpu.CompilerParams(
            use_tc_tiling_on_sc=False, skip_device_barrier=True
        ),
    )
    def emb(idx_hbm, w_hbm, out_hbm, idx_v, rows_v, sem0, sem1):
        sems = (sem0, sem1)
        wid = lax.axis_index("s") * mesh.num_cores + lax.axis_index("c")
        base = wid * per_w
        pltpu.sync_copy(idx_hbm.at[wid], idx_v)
        for b in range(nbuf):
            pltpu.async_copy(w_hbm.at[idx_v.at[b]], rows_v.at[b], sems[b])

        @pl.loop(0, n_ch // nbuf)
        def _(g):
            j0 = g * nbuf
            for b in range(nbuf):
                j = j0 + b
                pltpu.make_async_copy(
                    w_hbm.at[idx_v.at[b]], rows_v.at[b], sems[b]
                ).wait()
                pltpu.sync_copy(
                    rows_v.at[b],
                    out_hbm.at[pl.ds(base + j * ch, ch), pl.ds(0, d)],
                )
                nj = j + nbuf

                @pl.when(nj < n_ch)
                def _():
                    pltpu.async_copy(
                        w_hbm.at[idx_v.at[nj]], rows_v.at[b], sems[b]
                    )

    return emb(idx_grouped, weight)


def kernel(idx, weight):
    b, t = idx.shape
    v, d = weight.shape
    n = b * t
    nw = 32
    ch = 128
    per_w = n // nw
    n_ch = per_w // ch
    assert per_w * nw == n and n_ch * ch == per_w

    idx_grouped = idx.reshape(nw, n_ch, ch).astype(jnp.int32)
    rows = _emb_lookup(idx_grouped, weight, nw=nw, n_ch=n_ch, ch=ch, d=d)
    return rows[:, :d].reshape(b, t, d)
